# baseline (speedup 1.0000x reference)
import functools

import jax
import jax.numpy as jnp
from jax.experimental import pallas as pl
from jax.experimental.pallas import tpu as pltpu

_RPB = 128


@functools.lru_cache(maxsize=None)
def _build(batch, emb):
    grid = (batch // _RPB,)

    def body(uidx_s, iidx_s, ut_ref, it_ref, uout_ref, iout_ref,
             usem, isem):
        step = pl.program_id(0)
        for r in range(_RPB):
            ui = uidx_s[step * _RPB + r]
            ii = iidx_s[step * _RPB + r]
            pltpu.async_copy(ut_ref.at[ui], uout_ref.at[r], usem)
            pltpu.async_copy(it_ref.at[ii], iout_ref.at[r], isem)
        for r in range(_RPB):
            pltpu.make_async_copy(ut_ref.at[0], uout_ref.at[0], usem).wait()
            pltpu.make_async_copy(it_ref.at[0], iout_ref.at[0], isem).wait()

    grid_spec = pltpu.PrefetchScalarGridSpec(
        num_scalar_prefetch=2,
        grid=grid,
        in_specs=[
            pl.BlockSpec(memory_space=pl.ANY),
            pl.BlockSpec(memory_space=pl.ANY),
        ],
        out_specs=[
            pl.BlockSpec((_RPB, emb), lambda i, u, it: (i, 0)),
            pl.BlockSpec((_RPB, emb), lambda i, u, it: (i, 0)),
        ],
        scratch_shapes=[pltpu.SemaphoreType.DMA, pltpu.SemaphoreType.DMA],
    )
    return pl.pallas_call(
        body,
        grid_spec=grid_spec,
        out_shape=(
            jax.ShapeDtypeStruct((batch, emb), jnp.float32),
            jax.ShapeDtypeStruct((batch, emb), jnp.float32),
        ),
    )


def kernel(user, item, user_table, item_table):
    batch, = user.shape
    _, emb = user_table.shape
    return _build(batch, emb)(user, item, user_table, item_table)
